# baseline (speedup 1.0000x reference)
import jax
import jax.numpy as jnp
from jax.experimental import pallas as pl
from jax.experimental.pallas import tpu as pltpu

D = 768
RANK = 192
M_BLK = 2048
NBUF_IN = 4
NBUF_OUT = 3


def _fused_lowrank_kernel(x_hbm, u_ref, v_ref, o_hbm,
                          w_vmem, x_vmem, o_vmem, in_sems, out_sems):
    m = x_hbm.shape[0]
    num = m // M_BLK

    def in_copy(i, slot):
        return pltpu.make_async_copy(
            x_hbm.at[pl.ds(i * M_BLK, M_BLK), :], x_vmem.at[slot],
            in_sems.at[slot])

    def out_copy(i, slot):
        return pltpu.make_async_copy(
            o_vmem.at[slot], o_hbm.at[pl.ds(i * M_BLK, M_BLK), :],
            out_sems.at[slot])

    for k in range(min(NBUF_IN - 1, num)):
        in_copy(k, k).start()

    w_vmem[...] = jnp.dot(u_ref[...], v_ref[...],
                          preferred_element_type=jnp.float32)

    def loop(i, carry):
        in_slot = jax.lax.rem(i, NBUF_IN)
        out_slot = jax.lax.rem(i, NBUF_OUT)
        nxt = i + NBUF_IN - 1

        @pl.when(nxt < num)
        def _():
            in_copy(nxt, jax.lax.rem(nxt, NBUF_IN)).start()

        in_copy(i, in_slot).wait()

        @pl.when(i >= NBUF_OUT)
        def _():
            out_copy(i - NBUF_OUT, out_slot).wait()

        o_vmem[out_slot] = jnp.dot(x_vmem[in_slot], w_vmem[...],
                                   preferred_element_type=jnp.float32)
        out_copy(i, out_slot).start()
        return carry

    jax.lax.fori_loop(0, num, loop, 0)

    for i in range(max(num - NBUF_OUT, 0), num):
        out_copy(i, i % NBUF_OUT).wait()


def kernel(x, U, V):
    b, s, d = x.shape
    m = b * s
    x2 = x.reshape(m, d)
    out = pl.pallas_call(
        _fused_lowrank_kernel,
        in_specs=[
            pl.BlockSpec(memory_space=pltpu.MemorySpace.HBM),
            pl.BlockSpec(memory_space=pltpu.MemorySpace.VMEM),
            pl.BlockSpec(memory_space=pltpu.MemorySpace.VMEM),
        ],
        out_specs=pl.BlockSpec(memory_space=pltpu.MemorySpace.HBM),
        out_shape=jax.ShapeDtypeStruct((m, d), x.dtype),
        scratch_shapes=[
            pltpu.VMEM((D, D), jnp.float32),
            pltpu.VMEM((NBUF_IN, M_BLK, D), jnp.float32),
            pltpu.VMEM((NBUF_OUT, M_BLK, D), jnp.float32),
            pltpu.SemaphoreType.DMA((NBUF_IN,)),
            pltpu.SemaphoreType.DMA((NBUF_OUT,)),
        ],
    )(x2, U, V)
    return out.reshape(b, s, d)
